# Initial kernel scaffold; baseline (speedup 1.0000x reference)
import jax
import jax.numpy as jnp
from jax.experimental import pallas as pl


def kernel(xyz, params):
    raise NotImplementedError("write your pallas kernel here")


# speedup vs baseline: 4.4063x; 4.4063x over previous
import functools
import jax
import jax.numpy as jnp
from jax import lax
from jax.experimental import pallas as pl
from jax.experimental.pallas import tpu as pltpu

_EPS = 1e-5


def _dot_bf16(a, b):
    return jnp.dot(a.astype(jnp.bfloat16), b.astype(jnp.bfloat16),
                   preferred_element_type=jnp.float32)


def _dot(a, b):
    return _dot_bf16(a, b)


def _dot_hi(a, b):
    return jnp.dot(a, b, precision=lax.Precision.HIGHEST,
                   preferred_element_type=jnp.float32)


def _sqdist(nbT, xb):
    d = -2.0 * _dot_bf16(nbT, xb)
    d = d + jnp.sum(nbT * nbT, axis=1, keepdims=True)
    d = d + jnp.sum(xb * xb, axis=0, keepdims=True)
    return d


def _fps_body(npoint, xyz_ref, out_ref):
    x = xyz_ref[0, :, :]
    y = xyz_ref[1, :, :]
    z = xyz_ref[2, :, :]
    B, N = x.shape
    iota_n = lax.broadcasted_iota(jnp.int32, (B, N), 1)
    iota_s = lax.broadcasted_iota(jnp.int32, (B, npoint), 1)

    def body(i, state):
        dist, far, ox, oy, oz = state
        selm = iota_n == far
        cx = jnp.sum(jnp.where(selm, x, 0.0), axis=1, keepdims=True)
        cy = jnp.sum(jnp.where(selm, y, 0.0), axis=1, keepdims=True)
        cz = jnp.sum(jnp.where(selm, z, 0.0), axis=1, keepdims=True)
        put = iota_s == i
        ox = jnp.where(put, cx, ox)
        oy = jnp.where(put, cy, oy)
        oz = jnp.where(put, cz, oz)
        d = (x - cx) ** 2 + (y - cy) ** 2 + (z - cz) ** 2
        dist = jnp.minimum(dist, d)
        m = jnp.max(dist, axis=1, keepdims=True)
        far = jnp.min(jnp.where(dist == m, iota_n, N), axis=1, keepdims=True)
        return dist, far, ox, oy, oz

    dist0 = jnp.full((B, N), 1e10, jnp.float32)
    far0 = jnp.zeros((B, 1), jnp.int32)
    o0 = jnp.zeros((B, npoint), jnp.float32)
    _, _, ox, oy, oz = lax.fori_loop(0, npoint, body, (dist0, far0, o0, o0, o0))
    out_ref[0, :, :] = ox
    out_ref[1, :, :] = oy
    out_ref[2, :, :] = oz


def _fps(xyz_cbn, npoint):
    three, B, N = xyz_cbn.shape
    return pl.pallas_call(
        functools.partial(_fps_body, npoint),
        out_shape=jax.ShapeDtypeStruct((3, B, npoint), jnp.float32),
    )(xyz_cbn)


def _group_stats_write(sums_ref, y4d):
    C = y4d.shape[-1]
    s1 = jnp.sum(y4d, axis=(0, 1, 2))[None, :]
    blk = jnp.concatenate([s1, jnp.zeros((7, C), jnp.float32)], axis=0)
    first = (pl.program_id(0) == 0) & (pl.program_id(1) == 0)

    @pl.when(first)
    def _():
        sums_ref[...] = blk

    @pl.when(jnp.logical_not(first))
    def _():
        sums_ref[...] += blk


def _group1_body(nsample, radius2, xyz_cn_ref, xyz_nc_ref, new_ref, wx_ref,
                 b_ref, y_ref, sums_ref, key_ref):
    xb = xyz_cn_ref[0]
    xbT = xyz_nc_ref[0]
    nbT = new_ref[0]
    St, N = key_ref.shape
    d = _sqdist(nbT, xb)
    iota = lax.broadcasted_iota(jnp.int32, (St, N), 1)
    key0 = jnp.where(d > radius2, N, iota)
    key_ref[...] = key0
    first = jnp.min(key0, axis=1, keepdims=True)

    def body(k, _):
        key = key_ref[...]
        m = jnp.min(key, axis=1, keepdims=True)
        sel = jnp.where(m < N, m, first)
        oh = (iota == jnp.minimum(sel, N - 1)).astype(jnp.float32)
        pts = _dot_hi(oh, xbT)
        centered = pts - nbT
        yk = _dot(centered, wx_ref[...]) + b_ref[...]
        y_ref[0, pl.ds(k, 1)] = yk[None]
        key_ref[...] = jnp.where(key == m, N, key)
        return 0

    lax.fori_loop(0, nsample, body, 0)
    _group_stats_write(sums_ref, y_ref[...])


def _group2_body(nsample, radius2, xyz_cn_ref, xyz_nc_ref, new_ref, feat_ref,
                 wx_ref, wf_ref, b_ref, y_ref, sums_ref, key_ref):
    xb = xyz_cn_ref[0]
    xbT = xyz_nc_ref[0]
    nbT = new_ref[0]
    fb = feat_ref[0]
    St, N = key_ref.shape
    d = _sqdist(nbT, xb)
    iota = lax.broadcasted_iota(jnp.int32, (St, N), 1)
    key0 = jnp.where(d > radius2, N, iota)
    key_ref[...] = key0
    first = jnp.min(key0, axis=1, keepdims=True)

    def body(k, _):
        key = key_ref[...]
        m = jnp.min(key, axis=1, keepdims=True)
        sel = jnp.where(m < N, m, first)
        oh = (iota == jnp.minimum(sel, N - 1)).astype(jnp.float32)
        pts = _dot_hi(oh, xbT)
        centered = pts - nbT
        fk = _dot_hi(oh, fb)
        yk = _dot(centered, wx_ref[...])
        yk = yk + _dot(fk, wf_ref[...])
        yk = yk + b_ref[...]
        y_ref[0, pl.ds(k, 1)] = yk[None]
        key_ref[...] = jnp.where(key == m, N, key)
        return 0

    lax.fori_loop(0, nsample, body, 0)
    _group_stats_write(sums_ref, y_ref[...])


def _group1(xyz_cn, xyz_nc, new_nc, wx, b, radius, nsample, s_tile):
    B, three, N = xyz_cn.shape
    S = new_nc.shape[1]
    C = wx.shape[1]
    grid = (B, S // s_tile)
    return pl.pallas_call(
        functools.partial(_group1_body, nsample, radius * radius),
        grid=grid,
        in_specs=[
            pl.BlockSpec((1, 3, N), lambda b_, s_: (b_, 0, 0)),
            pl.BlockSpec((1, N, 3), lambda b_, s_: (b_, 0, 0)),
            pl.BlockSpec((1, s_tile, 3), lambda b_, s_: (b_, s_, 0)),
            pl.BlockSpec((3, C), lambda b_, s_: (0, 0)),
            pl.BlockSpec((1, C), lambda b_, s_: (0, 0)),
        ],
        out_specs=[
            pl.BlockSpec((1, nsample, s_tile, C),
                         lambda b_, s_: (b_, 0, s_, 0)),
            pl.BlockSpec((8, C), lambda b_, s_: (0, 0)),
        ],
        out_shape=[
            jax.ShapeDtypeStruct((B, nsample, S, C), jnp.float32),
            jax.ShapeDtypeStruct((8, C), jnp.float32),
        ],
        scratch_shapes=[pltpu.VMEM((s_tile, N), jnp.int32)],
    )(xyz_cn, xyz_nc, new_nc, wx, b)


def _group2(xyz_cn, xyz_nc, new_nc, feats, wx, wf, b, radius, nsample, s_tile):
    B, three, N = xyz_cn.shape
    S = new_nc.shape[1]
    Cf = feats.shape[2]
    C = wx.shape[1]
    grid = (B, S // s_tile)
    return pl.pallas_call(
        functools.partial(_group2_body, nsample, radius * radius),
        grid=grid,
        in_specs=[
            pl.BlockSpec((1, 3, N), lambda b_, s_: (b_, 0, 0)),
            pl.BlockSpec((1, N, 3), lambda b_, s_: (b_, 0, 0)),
            pl.BlockSpec((1, s_tile, 3), lambda b_, s_: (b_, s_, 0)),
            pl.BlockSpec((1, N, Cf), lambda b_, s_: (b_, 0, 0)),
            pl.BlockSpec((3, C), lambda b_, s_: (0, 0)),
            pl.BlockSpec((Cf, C), lambda b_, s_: (0, 0)),
            pl.BlockSpec((1, C), lambda b_, s_: (0, 0)),
        ],
        out_specs=[
            pl.BlockSpec((1, nsample, s_tile, C),
                         lambda b_, s_: (b_, 0, s_, 0)),
            pl.BlockSpec((8, C), lambda b_, s_: (0, 0)),
        ],
        out_shape=[
            jax.ShapeDtypeStruct((B, nsample, S, C), jnp.float32),
            jax.ShapeDtypeStruct((8, C), jnp.float32),
        ],
        scratch_shapes=[pltpu.VMEM((s_tile, N), jnp.int32)],
    )(xyz_cn, xyz_nc, new_nc, feats, wx, wf, b)


def _norm_relu(y, mean, var, gamma, beta):
    return jnp.maximum(gamma * (y - mean) / jnp.sqrt(var + _EPS) + beta, 0.0)


def _mlp_body(nstat, yp_ref, sums_in_ref, var_in_ref, g_ref, bt_ref, w_ref,
              b_ref, y_ref, sums_ref):
    mean = sums_in_ref[0:1, :]
    var = var_in_ref[0:1, :]
    x = _norm_relu(yp_ref[...], mean, var, g_ref[...], bt_ref[...])
    y = _dot(x, w_ref[...]) + b_ref[...]
    y_ref[...] = y
    C = y.shape[1]
    p1 = jnp.sum(y, axis=0, keepdims=True)
    blk = jnp.concatenate([p1, jnp.zeros((7, C), jnp.float32)], axis=0)

    @pl.when(pl.program_id(0) == 0)
    def _():
        sums_ref[...] = blk

    @pl.when(pl.program_id(0) != 0)
    def _():
        sums_ref[...] += blk


def _var_body(nstat, y_ref, sums_ref, var_ref):
    mean = sums_ref[0:1, :] / nstat
    d = y_ref[...] - mean
    C = d.shape[1]
    p = jnp.sum(d * d, axis=0, keepdims=True)
    blk = jnp.concatenate([p, jnp.zeros((7, C), jnp.float32)], axis=0)

    @pl.when(pl.program_id(0) == 0)
    def _():
        var_ref[...] = blk

    @pl.when(pl.program_id(0) != 0)
    def _():
        var_ref[...] += blk


def _var_pass(y, sums, nstat, tile):
    R, C = y.shape
    return pl.pallas_call(
        functools.partial(_var_body, nstat),
        grid=(R // tile,),
        in_specs=[
            pl.BlockSpec((tile, C), lambda i: (i, 0)),
            pl.BlockSpec((8, C), lambda i: (0, 0)),
        ],
        out_specs=pl.BlockSpec((8, C), lambda i: (0, 0)),
        out_shape=jax.ShapeDtypeStruct((8, C), jnp.float32),
    )(y, sums)


def _mlp_layer(yp, sums_in, var_in, gamma, beta, w, b, nstat, tile):
    R, Cin = yp.shape
    Cout = w.shape[1]
    return pl.pallas_call(
        functools.partial(_mlp_body, nstat),
        grid=(R // tile,),
        in_specs=[
            pl.BlockSpec((tile, Cin), lambda i: (i, 0)),
            pl.BlockSpec((8, Cin), lambda i: (0, 0)),
            pl.BlockSpec((8, Cin), lambda i: (0, 0)),
            pl.BlockSpec((1, Cin), lambda i: (0, 0)),
            pl.BlockSpec((1, Cin), lambda i: (0, 0)),
            pl.BlockSpec((Cin, Cout), lambda i: (0, 0)),
            pl.BlockSpec((1, Cout), lambda i: (0, 0)),
        ],
        out_specs=[
            pl.BlockSpec((tile, Cout), lambda i: (i, 0)),
            pl.BlockSpec((8, Cout), lambda i: (0, 0)),
        ],
        out_shape=[
            jax.ShapeDtypeStruct((R, Cout), jnp.float32),
            jax.ShapeDtypeStruct((8, Cout), jnp.float32),
        ],
    )(yp, sums_in, var_in, gamma, beta, w, b)


def _pool_body(nstat, y_ref, sums_ref, var_ref, g_ref, bt_ref, out_ref):
    mean = sums_ref[0:1, :]
    var = var_ref[0:1, :]
    C = mean.shape[1]
    y = y_ref[0]
    x = _norm_relu(y, mean.reshape(1, 1, C), var.reshape(1, 1, C),
                   g_ref[...].reshape(1, 1, C), bt_ref[...].reshape(1, 1, C))
    out_ref[0] = jnp.max(x, axis=0)


def _pool(y4d, sums_in, var_in, gamma, beta, nstat):
    B, n, S, C = y4d.shape
    return pl.pallas_call(
        functools.partial(_pool_body, nstat),
        grid=(B,),
        in_specs=[
            pl.BlockSpec((1, n, S, C), lambda b_: (b_, 0, 0, 0)),
            pl.BlockSpec((8, C), lambda b_: (0, 0)),
            pl.BlockSpec((8, C), lambda b_: (0, 0)),
            pl.BlockSpec((1, C), lambda b_: (0, 0)),
            pl.BlockSpec((1, C), lambda b_: (0, 0)),
        ],
        out_specs=pl.BlockSpec((1, S, C), lambda b_: (b_, 0, 0)),
        out_shape=jax.ShapeDtypeStruct((B, S, C), jnp.float32),
    )(y4d, sums_in, var_in, gamma, beta)


def _lin2_body(x_ref, f_ref, wx_ref, wf_ref, b_ref, y_ref):
    y = _dot(x_ref[...], wx_ref[...])
    y_ref[...] = y + _dot(f_ref[...], wf_ref[...]) + b_ref[...]


def _lin2(x, f, wx, wf, b):
    R = x.shape[0]
    C = wx.shape[1]
    return pl.pallas_call(
        _lin2_body,
        out_shape=jax.ShapeDtypeStruct((R, C), jnp.float32),
    )(x, f, wx, wf, b)


def _maxpool_body(y_ref, out_ref):
    out_ref[...] = jnp.max(y_ref[...], axis=1)


def _maxpool(y3d):
    B, n, C = y3d.shape
    return pl.pallas_call(
        _maxpool_body,
        out_shape=jax.ShapeDtypeStruct((B, C), jnp.float32),
    )(y3d)


def _finish_body(y_ref, m_ref, v_ref, g_ref, t_ref, out_ref):
    out_ref[...] = _norm_relu(y_ref[...], m_ref[0:1, :], v_ref[0:1, :],
                              g_ref[...], t_ref[...])


def _finish(y, m, v, gamma, beta):
    return pl.pallas_call(
        _finish_body,
        out_shape=jax.ShapeDtypeStruct(y.shape, jnp.float32),
    )(y, m, v, gamma, beta)


def _cols_stats(y2d, C):
    yb = lax.optimization_barrier(y2d)
    m, v = lax.optimization_barrier(
        (jnp.mean(yb, axis=0), jnp.var(yb, axis=0)))
    return (jnp.broadcast_to(m[None], (8, C)),
            jnp.broadcast_to(v[None], (8, C)))


def _tail(x_flat, f_flat, B, S, sa3, fc1, fc2):
    (w1, b1, g1, t1), (w2, b2, g2, t2), (w3, b3, _, _) = sa3
    fw1, fb1, fg1, ft1 = fc1
    fw2, fb2, fg2, ft2 = fc2
    R = x_flat.shape[0]
    y1 = _lin2(x_flat, f_flat, w1[:, :3].T, w1[:, 3:].T, b1[None])
    m1, v1 = _bn_stats(y1.reshape(B, S, 1, -1), w1.shape[0])
    y2, _ = _mlp_layer(y1, m1, v1, g1[None], t1[None], w2.T, b2[None],
                       R, tile=R)
    m2, v2 = _bn_stats(y2.reshape(B, S, 1, -1), w2.shape[0])
    y3, _ = _mlp_layer(y2, m2, v2, g2[None], t2[None], w3.T, b3[None],
                       R, tile=R)
    pooled = _maxpool(y3.reshape(B, S, -1))
    z1 = _lin2(pooled, jnp.zeros((B, 1), jnp.float32),
               fw1.T, jnp.zeros((1, fw1.shape[0]), jnp.float32), fb1[None])
    n1, o1 = _cols_stats(z1, fw1.shape[0])
    z2, _ = _mlp_layer(z1, n1, o1, fg1[None], ft1[None], fw2.T, fb2[None],
                       B, tile=B)
    n2, o2 = _cols_stats(z2, fw2.shape[0])
    return _finish(z2, n2, o2, fg2[None], ft2[None])


def _bn_stats(y4d, C):
    yt = lax.optimization_barrier(jnp.transpose(y4d, (0, 2, 1, 3)))
    m, v = lax.optimization_barrier(
        (jnp.mean(yt, axis=(0, 1, 2)), jnp.var(yt, axis=(0, 1, 2))))
    return (jnp.broadcast_to(m[None], (8, C)),
            jnp.broadcast_to(v[None], (8, C)))


def kernel(xyz, params):
    B, _, N = xyz.shape
    xyz_cn = xyz
    xyz_cbn = jnp.transpose(xyz, (1, 0, 2))
    xyz_nc = jnp.transpose(xyz, (0, 2, 1))

    sa1, sa2, sa3 = params['sa1'], params['sa2'], params['sa3']

    S1, K1 = 512, 32
    new1_cbs = _fps(xyz_cbn, S1)
    new1_nc = jnp.transpose(new1_cbs, (1, 2, 0))
    (w1, b1, g1, t1), (w2, b2, g2, t2), (w3, b3, g3, t3) = sa1
    y1, _ = _group1(xyz_cn, xyz_nc, new1_nc, w1.T, b1[None],
                    0.2, K1, s_tile=256)
    R1 = B * K1 * S1
    m1, v1 = _bn_stats(y1, 64)
    y2f, _ = _mlp_layer(y1.reshape(R1, -1), m1, v1, g1[None], t1[None],
                        w2.T, b2[None], R1, tile=4096)
    m2, v2 = _bn_stats(y2f.reshape(B, K1, S1, -1), 64)
    y3f, _ = _mlp_layer(y2f, m2, v2, g2[None], t2[None], w3.T, b3[None],
                        R1, tile=4096)
    y3 = y3f.reshape(B, K1, S1, -1)
    m3, v3 = _bn_stats(y3, 128)
    l1_points = _pool(y3, m3, v3, g3[None], t3[None], R1)

    S2, K2 = 128, 64
    new2_cbs = _fps(new1_cbs, S2)
    new2_nc = jnp.transpose(new2_cbs, (1, 2, 0))
    xyz2_cn = jnp.transpose(new1_cbs, (1, 0, 2))
    (u1, c1, e1, r1), (u2, c2, e2, r2), (u3, c3, e3, r3) = sa2
    z1, q1 = _group2(xyz2_cn, new1_nc, new2_nc, l1_points,
                     u1[:, :3].T, u1[:, 3:].T, c1[None], 0.4, K2, s_tile=128)
    R2 = B * K2 * S2
    n1, o1 = _bn_stats(z1, 128)
    z2f, _ = _mlp_layer(z1.reshape(R2, -1), n1, o1, e1[None], r1[None],
                        u2.T, c2[None], R2, tile=4096)
    n2, o2 = _bn_stats(z2f.reshape(B, K2, S2, -1), 128)
    z3f, _ = _mlp_layer(z2f, n2, o2, e2[None], r2[None], u3.T, c3[None],
                        R2, tile=4096)
    z3 = z3f.reshape(B, K2, S2, -1)
    n3, o3 = _bn_stats(z3, 256)
    l2_points = _pool(z3, n3, o3, e3[None], r3[None], R2)

    x_flat = new2_nc.reshape(B * S2, 3)
    f_flat = l2_points.reshape(B * S2, -1)
    return _tail(x_flat, f_flat, B, S2, sa3, params['fc1'], params['fc2'])
